# baseline (speedup 1.0000x reference)
import functools

import numpy as np

import jax
import jax.numpy as jnp
from jax import lax
from jax.experimental import pallas as pl
from jax.experimental.pallas import tpu as pltpu

_B, _S, _H, _A = 16384, 480, 128, 1000
_BM = 256


def _np_gumbel(seed: int, shape) -> np.ndarray:
    def rotl(x, r):
        return (x << np.uint32(r)) | (x >> np.uint32(32 - r))

    n = int(np.prod(shape))
    k1 = np.uint32((seed >> 32) & 0xffffffff)
    k2 = np.uint32(seed & 0xffffffff)
    idx = np.arange(n, dtype=np.uint64)
    x0 = (idx >> np.uint64(32)).astype(np.uint32)
    x1 = (idx & np.uint64(0xffffffff)).astype(np.uint32)
    rotations = [(13, 15, 26, 6), (17, 29, 16, 24)]
    ks = [k1, k2, k1 ^ k2 ^ np.uint32(0x1BD11BDA)]
    x0 = x0 + ks[0]
    x1 = x1 + ks[1]
    for i in range(5):
        for r in rotations[i % 2]:
            x0 = x0 + x1
            x1 = rotl(x1, r)
            x1 = x0 ^ x1
        x0 = x0 + ks[(i + 1) % 3]
        x1 = x1 + ks[(i + 2) % 3] + np.uint32(i + 1)
    bits = x0 ^ x1
    fb = (bits >> np.uint32(9)) | np.uint32(0x3f800000)
    floats = fb.view(np.float32) - np.float32(1.0)
    tiny = np.float32(np.finfo(np.float32).tiny)
    u = np.maximum(tiny, floats * (np.float32(1.0) - tiny) + tiny)
    return (-np.log(-np.log(u))).astype(np.float32).reshape(shape)


_GUMBEL = _np_gumbel(42, (_B, _A))

_NEG_INF = float("-inf")


def _argmax_first(x, iota, big):
    vmax = jnp.max(x, axis=-1, keepdims=True)
    return jnp.min(jnp.where(x == vmax, iota, big), axis=-1)


def _fused_body(s_ref, ps_ref, legal_ref, gum_ref, greedy_ref,
                W1_ref, b1_ref, W2_ref, b2_ref,
                V1_ref, Vb1_ref, V2_ref, Vb2_ref,
                lp_ref, act_ref, val_ref):
    h = jnp.maximum(
        jnp.dot(s_ref[...], W1_ref[...], preferred_element_type=jnp.float32)
        + b1_ref[...], 0.0)
    logits = (jnp.dot(h, W2_ref[...], preferred_element_type=jnp.float32)
              + b2_ref[...])
    shifted = logits - jnp.max(logits, axis=-1, keepdims=True)
    lp = shifted - jnp.log(jnp.sum(jnp.exp(shifted), axis=-1, keepdims=True))
    lp_ref[...] = lp

    noise_gate = 1.0 - greedy_ref[...].astype(jnp.float32)[:, None]
    score = jnp.where(legal_ref[...] > 0.0,
                      lp + gum_ref[...] * noise_gate, _NEG_INF)
    iota = lax.broadcasted_iota(jnp.int32, score.shape, 1)
    act_ref[...] = _argmax_first(score, iota, jnp.int32(_A)).astype(jnp.int32)

    vh = jnp.maximum(
        jnp.dot(ps_ref[...], V1_ref[...], preferred_element_type=jnp.float32)
        + Vb1_ref[...], 0.0)
    val_ref[...] = (jnp.dot(vh, V2_ref[...], preferred_element_type=jnp.float32)
                    + Vb2_ref[...])[:, 0]


@functools.partial(jax.jit, donate_argnums=())
def kernel(s, perfect_s, legal_actions, greedy, W1, b1, W2, b2, V1, Vb1, V2, Vb2):
    nb = _B // _BM
    row = lambda i: (i, 0)
    full = lambda i: (0, 0)
    full1 = lambda i: (0,)

    grid_spec = pl.GridSpec(
        grid=(nb,),
        in_specs=[
            pl.BlockSpec((_BM, _S), row),
            pl.BlockSpec((_BM, _S), row),
            pl.BlockSpec((_BM, _A), row),
            pl.BlockSpec((_BM, _A), row),
            pl.BlockSpec((_BM,), lambda i: (i,)),
            pl.BlockSpec((_S, _H), full),
            pl.BlockSpec((_H,), full1),
            pl.BlockSpec((_H, _A), full),
            pl.BlockSpec((_A,), full1),
            pl.BlockSpec((_S, _H), full),
            pl.BlockSpec((_H,), full1),
            pl.BlockSpec((_H, 1), full),
            pl.BlockSpec((1,), full1),
        ],
        out_specs=[
            pl.BlockSpec((_BM, _A), row),
            pl.BlockSpec((_BM,), lambda i: (i,)),
            pl.BlockSpec((_BM,), lambda i: (i,)),
        ],
    )
    lp, act, val = pl.pallas_call(
        _fused_body,
        grid_spec=grid_spec,
        out_shape=[
            jax.ShapeDtypeStruct((_B, _A), jnp.float32),
            jax.ShapeDtypeStruct((_B,), jnp.int32),
            jax.ShapeDtypeStruct((_B,), jnp.float32),
        ],
        compiler_params=pltpu.CompilerParams(
            dimension_semantics=("parallel",),
        ),
    )(s, perfect_s, legal_actions, _GUMBEL, greedy,
      W1, b1, W2, b2, V1, Vb1, V2, Vb2)
    return act, lp, val


# speedup vs baseline: 1.0831x; 1.0831x over previous
import functools

import numpy as np

import jax
import jax.numpy as jnp
from jax import lax
from jax.experimental import pallas as pl
from jax.experimental.pallas import tpu as pltpu

_B, _S, _H, _A = 16384, 480, 128, 1000
_BM = 1024


def _np_gumbel(seed: int, shape) -> np.ndarray:
    def rotl(x, r):
        return (x << np.uint32(r)) | (x >> np.uint32(32 - r))

    n = int(np.prod(shape))
    k1 = np.uint32((seed >> 32) & 0xffffffff)
    k2 = np.uint32(seed & 0xffffffff)
    idx = np.arange(n, dtype=np.uint64)
    x0 = (idx >> np.uint64(32)).astype(np.uint32)
    x1 = (idx & np.uint64(0xffffffff)).astype(np.uint32)
    rotations = [(13, 15, 26, 6), (17, 29, 16, 24)]
    ks = [k1, k2, k1 ^ k2 ^ np.uint32(0x1BD11BDA)]
    x0 = x0 + ks[0]
    x1 = x1 + ks[1]
    for i in range(5):
        for r in rotations[i % 2]:
            x0 = x0 + x1
            x1 = rotl(x1, r)
            x1 = x0 ^ x1
        x0 = x0 + ks[(i + 1) % 3]
        x1 = x1 + ks[(i + 2) % 3] + np.uint32(i + 1)
    bits = x0 ^ x1
    fb = (bits >> np.uint32(9)) | np.uint32(0x3f800000)
    floats = fb.view(np.float32) - np.float32(1.0)
    tiny = np.float32(np.finfo(np.float32).tiny)
    u = np.maximum(tiny, floats * (np.float32(1.0) - tiny) + tiny)
    return (-np.log(-np.log(u))).astype(np.float32).reshape(shape)


_GUMBEL = _np_gumbel(42, (_B, _A))

_NEG_INF = float("-inf")


def _argmax_first(x, iota, big):
    vmax = jnp.max(x, axis=-1, keepdims=True)
    return jnp.min(jnp.where(x == vmax, iota, big), axis=-1)


def _fused_body(s_ref, ps_ref, legal_ref, gum_ref, greedy_ref,
                W1_ref, b1_ref, W2_ref, b2_ref,
                V1_ref, Vb1_ref, V2_ref, Vb2_ref,
                lp_ref, act_ref, val_ref):
    h = jnp.maximum(
        jnp.dot(s_ref[...], W1_ref[...], preferred_element_type=jnp.float32)
        + b1_ref[...], 0.0)
    logits = (jnp.dot(h, W2_ref[...], preferred_element_type=jnp.float32)
              + b2_ref[...])
    shifted = logits - jnp.max(logits, axis=-1, keepdims=True)
    lp = shifted - jnp.log(jnp.sum(jnp.exp(shifted), axis=-1, keepdims=True))
    lp_ref[...] = lp

    noise_gate = 1.0 - greedy_ref[...].astype(jnp.float32)[:, None]
    score = jnp.where(legal_ref[...] > 0.0,
                      lp + gum_ref[...] * noise_gate, _NEG_INF)
    iota = lax.broadcasted_iota(jnp.int32, score.shape, 1)
    act_ref[...] = _argmax_first(score, iota, jnp.int32(_A)).astype(jnp.int32)

    vh = jnp.maximum(
        jnp.dot(ps_ref[...], V1_ref[...], preferred_element_type=jnp.float32)
        + Vb1_ref[...], 0.0)
    val_ref[...] = (jnp.dot(vh, V2_ref[...], preferred_element_type=jnp.float32)
                    + Vb2_ref[...])[:, 0]


@functools.partial(jax.jit, donate_argnums=())
def kernel(s, perfect_s, legal_actions, greedy, W1, b1, W2, b2, V1, Vb1, V2, Vb2):
    nb = _B // _BM
    row = lambda i: (i, 0)
    full = lambda i: (0, 0)
    full1 = lambda i: (0,)

    grid_spec = pl.GridSpec(
        grid=(nb,),
        in_specs=[
            pl.BlockSpec((_BM, _S), row),
            pl.BlockSpec((_BM, _S), row),
            pl.BlockSpec((_BM, _A), row),
            pl.BlockSpec((_BM, _A), row),
            pl.BlockSpec((_BM,), lambda i: (i,)),
            pl.BlockSpec((_S, _H), full),
            pl.BlockSpec((_H,), full1),
            pl.BlockSpec((_H, _A), full),
            pl.BlockSpec((_A,), full1),
            pl.BlockSpec((_S, _H), full),
            pl.BlockSpec((_H,), full1),
            pl.BlockSpec((_H, 1), full),
            pl.BlockSpec((1,), full1),
        ],
        out_specs=[
            pl.BlockSpec((_BM, _A), row),
            pl.BlockSpec((_BM,), lambda i: (i,)),
            pl.BlockSpec((_BM,), lambda i: (i,)),
        ],
    )
    lp, act, val = pl.pallas_call(
        _fused_body,
        grid_spec=grid_spec,
        out_shape=[
            jax.ShapeDtypeStruct((_B, _A), jnp.float32),
            jax.ShapeDtypeStruct((_B,), jnp.int32),
            jax.ShapeDtypeStruct((_B,), jnp.float32),
        ],
        compiler_params=pltpu.CompilerParams(
            dimension_semantics=("parallel",),
        ),
    )(s, perfect_s, legal_actions, _GUMBEL, greedy,
      W1, b1, W2, b2, V1, Vb1, V2, Vb2)
    return act, lp, val
